# baseline (speedup 1.0000x reference)
import functools

import jax
import jax.numpy as jnp
from jax import lax
from jax.experimental import pallas as pl
from jax.experimental.pallas import tpu as pltpu
from jax.experimental.pallas import tpu_sc as plsc

_NC = 2
_NS = 16
_L = 16
_NW = _NC * _NS

_M31 = 0x7FFFFFFF


def _to_ordered(v_f32):
    s = plsc.bitcast(v_f32, jnp.int32)
    return s ^ (jnp.right_shift(s, 31) & jnp.int32(_M31))


@functools.lru_cache(maxsize=None)
def _build(R, C):
    assert C % _L == 0 and R % _NW == 0
    nvec = C // _L
    rows_per_w = R // _NW
    UNROLL = 8
    assert nvec % UNROLL == 0

    mesh = plsc.VectorSubcoreMesh(
        core_axis_name="c", subcore_axis_name="s",
        num_cores=_NC, num_subcores=_NS)

    @functools.partial(
        pl.kernel,
        out_type=jax.ShapeDtypeStruct((R * C,), jnp.float32),
        mesh=mesh,
        compiler_params=pltpu.CompilerParams(needs_layout_passes=False),
        scratch_types=[
            pltpu.VMEM((2 * C,), jnp.float32),
            pltpu.VMEM((C,), jnp.float32),
            pltpu.VMEM((_L * 256,), jnp.int32),
            pltpu.VMEM((256,), jnp.int32),
            pltpu.VMEM((_L,), jnp.int32),
            pltpu.VMEM((_L,), jnp.int32),
            pltpu.SemaphoreType.DMA,
            pltpu.SemaphoreType.DMA,
            pltpu.SemaphoreType.DMA,
        ],
    )
    def sc_kernel(scores_hbm, k_hbm, out_hbm, rows_v, aux_v, hist_v, tot_v,
                  bs_v, k_v, sem_in0, sem_in1, sem_out):
        cid = lax.axis_index("c")
        sid = lax.axis_index("s")
        wid = sid * _NC + cid

        lane = lax.iota(jnp.int32, _L)
        lane_base = lane * jnp.int32(256)
        ones_i = jnp.ones((_L,), jnp.int32)
        zeros_i = jnp.zeros((_L,), jnp.int32)
        neg_f = jnp.full((_L,), -1e9, jnp.float32)

        pltpu.sync_copy(k_hbm, k_v)
        kk0 = k_v[pl.ds(0, _L)][0]

        bufs = (rows_v.at[pl.ds(0, C)], rows_v.at[pl.ds(C, C)])
        in_sems = (sem_in0, sem_in1)
        in_handles, out_handles = {}, {}

        def base_of(j):
            return (wid * jnp.int32(rows_per_w) + jnp.int32(j)) * jnp.int32(C)

        def start_in(j):
            in_handles[j] = pltpu.async_copy(
                scores_hbm.at[pl.ds(base_of(j), C)], bufs[j % 2],
                in_sems[j % 2])

        def zero_hist():
            @plsc.parallel_loop(0, 256, unroll=UNROLL)
            def _(i):
                hist_v[pl.ds(i * _L, _L)] = zeros_i

        lane_is0 = lane == zeros_i

        def select_digit(kk):
            @plsc.parallel_loop(0, 16, unroll=4)
            def _(t):
                tot = zeros_i
                for l in range(_L):
                    tot = tot + hist_v[pl.ds(l * 256 + t * _L, _L)]
                tot_v[pl.ds(t * _L, _L)] = tot
                s = jnp.full((_L,), jnp.sum(tot), jnp.int32)
                plsc.store_scatter(bs_v, [jnp.full((_L,), t, jnp.int32)], s,
                                   mask=lane_is0)

            bs = bs_v[pl.ds(0, _L)]
            sb = plsc.cumsum(lax.rev(bs, (0,)))
            blk_desc = jnp.int32(15) - lane
            cb = jnp.max(jnp.where(sb >= kk, blk_desc, jnp.int32(-1)))
            carry_above = jnp.sum(jnp.where(lane > cb, bs, zeros_i))
            tot_cb = tot_v[pl.ds(cb * _L, _L)]
            cs = plsc.cumsum(lax.rev(tot_cb, (0,))) + carry_above
            bin_desc = cb * jnp.int32(_L) + jnp.int32(15) - lane
            b_sel = jnp.max(jnp.where(cs >= kk, bin_desc, jnp.int32(-1)))
            bin_asc = cb * jnp.int32(_L) + lane
            cnt_gt = carry_above + jnp.sum(
                jnp.where(bin_asc > b_sel, tot_cb, zeros_i))
            return b_sel, kk - cnt_gt

        start_in(0)
        for j in range(rows_per_w):
            if j + 1 < rows_per_w:
                start_in(j + 1)
            in_handles[j].wait()
            boff = (j % 2) * C

            zero_hist()

            @plsc.parallel_loop(0, nvec, unroll=UNROLL)
            def _(i):
                v = rows_v[pl.ds(i * _L + boff, _L)]
                o = _to_ordered(v)
                d = jnp.right_shift(o, 24) + jnp.int32(128)
                plsc.addupdate_scatter(hist_v, [lane_base + d], ones_i)

            b0, kk = select_digit(kk0)
            prefix = b0 - jnp.int32(128)

            if j >= 1:
                out_handles[j - 1].wait()

            zero_hist()
            pref0 = prefix

            def compact_body(i, off):
                v = rows_v[pl.ds(i * _L + boff, _L)]
                o = _to_ordered(v)
                m = jnp.right_shift(o, 24) == pref0
                d = jnp.right_shift(o, 16) & jnp.int32(0xFF)
                plsc.addupdate_scatter(hist_v, [lane_base + d], ones_i,
                                       mask=m)
                pos = plsc.cumsum(jnp.where(m, ones_i, zeros_i))
                plsc.store_scatter(aux_v, [off + pos - 1],
                                   plsc.bitcast(o, jnp.float32), mask=m)
                return off + plsc.all_reduce_population_count(m)

            n1_splat = plsc.parallel_loop(
                0, nvec, unroll=UNROLL,
                carry=jnp.zeros((_L,), jnp.int32))(compact_body)
            n1 = n1_splat[0]
            b1, kk = select_digit(kk)
            prefix = prefix * jnp.int32(256) + b1

            n_steps = jnp.right_shift(n1 + jnp.int32(_L - 1), 4)
            for p in (2, 3):
                zero_hist()
                pref_s = prefix
                dsh = 24 - 8 * p

                @plsc.parallel_loop(0, n_steps, unroll=2)
                def _(i):
                    o = plsc.bitcast(aux_v[pl.ds(i * _L, _L)], jnp.int32)
                    valid = (i * jnp.int32(_L) + lane) < n1
                    m = valid & (jnp.right_shift(o, 32 - 8 * p) == pref_s)
                    if dsh:
                        d = jnp.right_shift(o, dsh) & jnp.int32(0xFF)
                    else:
                        d = o & jnp.int32(0xFF)
                    plsc.addupdate_scatter(hist_v, [lane_base + d], ones_i,
                                           mask=m)

                b_sel, kk = select_digit(kk)
                prefix = prefix * jnp.int32(256) + b_sel

            thr_o = jnp.full((_L,), prefix, jnp.int32)
            thr_s = thr_o ^ (jnp.right_shift(thr_o, 31) & jnp.int32(_M31))
            thr_f = plsc.bitcast(thr_s, jnp.float32)

            @plsc.parallel_loop(0, nvec, unroll=UNROLL)
            def _(i):
                v = rows_v[pl.ds(i * _L + boff, _L)]
                aux_v[pl.ds(i * _L, _L)] = jnp.where(v >= thr_f, v, neg_f)

            out_handles[j] = pltpu.async_copy(
                aux_v, out_hbm.at[pl.ds(base_of(j), C)], sem_out)

        out_handles[rows_per_w - 1].wait()

    return sc_kernel


def kernel(scores, k):
    R, C = scores.shape
    k_arr = jnp.full((_L,), k, dtype=jnp.int32)
    out_flat = _build(R, C)(scores.reshape(-1), k_arr)
    return out_flat.reshape(R, C)


# speedup vs baseline: 1.3875x; 1.3875x over previous
import functools

import jax
import jax.numpy as jnp
from jax import lax
from jax.experimental import pallas as pl
from jax.experimental.pallas import tpu as pltpu
from jax.experimental.pallas import tpu_sc as plsc

_NC = 2
_NS = 16
_L = 16
_NW = _NC * _NS

_M31 = 0x7FFFFFFF


def _to_ordered(v_f32):
    s = plsc.bitcast(v_f32, jnp.int32)
    return s ^ (jnp.right_shift(s, 31) & jnp.int32(_M31))


@functools.lru_cache(maxsize=None)
def _build(R, C):
    assert C % _L == 0 and R % _NW == 0
    nvec = C // _L
    rows_per_w = R // _NW
    UNROLL = 8
    assert nvec % UNROLL == 0

    mesh = plsc.VectorSubcoreMesh(
        core_axis_name="c", subcore_axis_name="s",
        num_cores=_NC, num_subcores=_NS)

    @functools.partial(
        pl.kernel,
        out_type=jax.ShapeDtypeStruct((R, C), jnp.float32),
        mesh=mesh,
        compiler_params=pltpu.CompilerParams(needs_layout_passes=False),
        scratch_types=[
            pltpu.VMEM((2 * C,), jnp.float32),
            pltpu.VMEM((C,), jnp.float32),
            pltpu.VMEM((_L * 256,), jnp.int32),
            pltpu.VMEM((256,), jnp.int32),
            pltpu.VMEM((_L,), jnp.int32),
            pltpu.VMEM((_L,), jnp.int32),
            pltpu.SemaphoreType.DMA,
            pltpu.SemaphoreType.DMA,
            pltpu.SemaphoreType.DMA,
        ],
    )
    def sc_kernel(scores_hbm, k_hbm, out_hbm, rows_v, aux_v, hist_v, tot_v,
                  bs_v, k_v, sem_in0, sem_in1, sem_out):
        cid = lax.axis_index("c")
        sid = lax.axis_index("s")
        wid = sid * _NC + cid

        lane = lax.iota(jnp.int32, _L)
        lane_base = lane * jnp.int32(256)
        ones_i = jnp.ones((_L,), jnp.int32)
        zeros_i = jnp.zeros((_L,), jnp.int32)
        neg_f = jnp.full((_L,), -1e9, jnp.float32)

        pltpu.sync_copy(k_hbm, k_v)
        kk0 = k_v[pl.ds(0, _L)][0]

        bufs = (rows_v.at[pl.ds(0, C)], rows_v.at[pl.ds(C, C)])
        in_sems = (sem_in0, sem_in1)
        in_handles, out_handles = {}, {}

        def base_of(j):
            return wid * jnp.int32(rows_per_w) + jnp.int32(j)

        def start_in(j):
            in_handles[j] = pltpu.async_copy(
                scores_hbm.at[base_of(j)], bufs[j % 2], in_sems[j % 2])

        def zero_hist():
            @plsc.parallel_loop(0, 256, unroll=UNROLL)
            def _(i):
                hist_v[pl.ds(i * _L, _L)] = zeros_i

        lane_is0 = lane == zeros_i

        def select_digit(kk):
            @plsc.parallel_loop(0, 16, unroll=4)
            def _(t):
                tot = zeros_i
                for l in range(_L):
                    tot = tot + hist_v[pl.ds(l * 256 + t * _L, _L)]
                tot_v[pl.ds(t * _L, _L)] = tot
                s = jnp.full((_L,), jnp.sum(tot), jnp.int32)
                plsc.store_scatter(bs_v, [jnp.full((_L,), t, jnp.int32)], s,
                                   mask=lane_is0)

            bs = bs_v[pl.ds(0, _L)]
            sb = plsc.cumsum(lax.rev(bs, (0,)))
            blk_desc = jnp.int32(15) - lane
            cb = jnp.max(jnp.where(sb >= kk, blk_desc, jnp.int32(-1)))
            carry_above = jnp.sum(jnp.where(lane > cb, bs, zeros_i))
            tot_cb = tot_v[pl.ds(cb * _L, _L)]
            cs = plsc.cumsum(lax.rev(tot_cb, (0,))) + carry_above
            bin_desc = cb * jnp.int32(_L) + jnp.int32(15) - lane
            b_sel = jnp.max(jnp.where(cs >= kk, bin_desc, jnp.int32(-1)))
            bin_asc = cb * jnp.int32(_L) + lane
            cnt_gt = carry_above + jnp.sum(
                jnp.where(bin_asc > b_sel, tot_cb, zeros_i))
            return b_sel, kk - cnt_gt

        start_in(0)
        for j in range(rows_per_w):
            if j + 1 < rows_per_w:
                start_in(j + 1)
            in_handles[j].wait()
            boff = (j % 2) * C

            zero_hist()

            @plsc.parallel_loop(0, nvec, unroll=UNROLL)
            def _(i):
                v = rows_v[pl.ds(i * _L + boff, _L)]
                o = _to_ordered(v)
                d = jnp.right_shift(o, 24) + jnp.int32(128)
                plsc.addupdate_scatter(hist_v, [lane_base + d], ones_i)

            b0, kk = select_digit(kk0)
            prefix = b0 - jnp.int32(128)

            if j >= 1:
                out_handles[j - 1].wait()

            zero_hist()
            pref0 = prefix

            def compact_body(i, off):
                v = rows_v[pl.ds(i * _L + boff, _L)]
                o = _to_ordered(v)
                m = jnp.right_shift(o, 24) == pref0
                d = jnp.right_shift(o, 16) & jnp.int32(0xFF)
                plsc.addupdate_scatter(hist_v, [lane_base + d], ones_i,
                                       mask=m)
                pos = plsc.cumsum(jnp.where(m, ones_i, zeros_i))
                plsc.store_scatter(aux_v, [off + pos - 1],
                                   plsc.bitcast(o, jnp.float32), mask=m)
                return off + plsc.all_reduce_population_count(m)

            n1_splat = plsc.parallel_loop(
                0, nvec, unroll=UNROLL,
                carry=jnp.zeros((_L,), jnp.int32))(compact_body)
            n1 = n1_splat[0]
            b1, kk = select_digit(kk)
            prefix = prefix * jnp.int32(256) + b1

            n_steps = jnp.right_shift(n1 + jnp.int32(_L - 1), 4)
            for p in (2, 3):
                zero_hist()
                pref_s = prefix
                dsh = 24 - 8 * p

                @plsc.parallel_loop(0, n_steps, unroll=2)
                def _(i):
                    o = plsc.bitcast(aux_v[pl.ds(i * _L, _L)], jnp.int32)
                    valid = (i * jnp.int32(_L) + lane) < n1
                    m = valid & (jnp.right_shift(o, 32 - 8 * p) == pref_s)
                    if dsh:
                        d = jnp.right_shift(o, dsh) & jnp.int32(0xFF)
                    else:
                        d = o & jnp.int32(0xFF)
                    plsc.addupdate_scatter(hist_v, [lane_base + d], ones_i,
                                           mask=m)

                b_sel, kk = select_digit(kk)
                prefix = prefix * jnp.int32(256) + b_sel

            thr_o = jnp.full((_L,), prefix, jnp.int32)
            thr_s = thr_o ^ (jnp.right_shift(thr_o, 31) & jnp.int32(_M31))
            thr_f = plsc.bitcast(thr_s, jnp.float32)

            @plsc.parallel_loop(0, nvec, unroll=UNROLL)
            def _(i):
                v = rows_v[pl.ds(i * _L + boff, _L)]
                aux_v[pl.ds(i * _L, _L)] = jnp.where(v >= thr_f, v, neg_f)

            out_handles[j] = pltpu.async_copy(
                aux_v, out_hbm.at[base_of(j)], sem_out)

        out_handles[rows_per_w - 1].wait()

    return sc_kernel


def kernel(scores, k):
    R, C = scores.shape
    k_arr = jnp.full((_L,), k, dtype=jnp.int32)
    return _build(R, C)(scores, k_arr)


# speedup vs baseline: 1.5206x; 1.0959x over previous
import functools

import jax
import jax.numpy as jnp
from jax import lax
from jax.experimental import pallas as pl
from jax.experimental.pallas import tpu as pltpu
from jax.experimental.pallas import tpu_sc as plsc

_NC = 2
_NS = 16
_L = 16
_NW = _NC * _NS

_M31 = 0x7FFFFFFF


def _to_ordered(v_f32):
    s = plsc.bitcast(v_f32, jnp.int32)
    return s ^ (jnp.right_shift(s, 31) & jnp.int32(_M31))


@functools.lru_cache(maxsize=None)
def _build(R, C):
    assert C % _L == 0 and R % _NW == 0
    nvec = C // _L
    rows_per_w = R // _NW
    UNROLL = 8
    assert nvec % UNROLL == 0

    mesh = plsc.VectorSubcoreMesh(
        core_axis_name="c", subcore_axis_name="s",
        num_cores=_NC, num_subcores=_NS)

    @functools.partial(
        pl.kernel,
        out_type=jax.ShapeDtypeStruct((R, C), jnp.float32),
        mesh=mesh,
        compiler_params=pltpu.CompilerParams(needs_layout_passes=False),
        scratch_types=[
            pltpu.VMEM((2 * C,), jnp.float32),
            pltpu.VMEM((C,), jnp.float32),
            pltpu.VMEM((_L * 256,), jnp.int32),
            pltpu.VMEM((256,), jnp.int32),
            pltpu.VMEM((_L,), jnp.int32),
            pltpu.VMEM((_L,), jnp.int32),
            pltpu.SemaphoreType.DMA,
            pltpu.SemaphoreType.DMA,
            pltpu.SemaphoreType.DMA,
        ],
    )
    def sc_kernel(scores_hbm, k_hbm, out_hbm, rows_v, aux_v, hist_v, tot_v,
                  bs_v, k_v, sem_in0, sem_in1, sem_out):
        cid = lax.axis_index("c")
        sid = lax.axis_index("s")
        wid = sid * _NC + cid

        lane = lax.iota(jnp.int32, _L)
        lane_base = lane * jnp.int32(256)
        ones_i = jnp.ones((_L,), jnp.int32)
        zeros_i = jnp.zeros((_L,), jnp.int32)
        neg_f = jnp.full((_L,), -1e9, jnp.float32)

        pltpu.sync_copy(k_hbm, k_v)
        kk0 = k_v[pl.ds(0, _L)][0]

        bufs = (rows_v.at[pl.ds(0, C)], rows_v.at[pl.ds(C, C)])
        in_sems = (sem_in0, sem_in1)
        in_handles, out_handles = {}, {}

        def base_of(j):
            return wid * jnp.int32(rows_per_w) + jnp.int32(j)

        def start_in(j):
            in_handles[j] = pltpu.async_copy(
                scores_hbm.at[base_of(j)], bufs[j % 2], in_sems[j % 2])

        def zero_hist():
            @plsc.parallel_loop(0, 256, unroll=UNROLL)
            def _(i):
                hist_v[pl.ds(i * _L, _L)] = zeros_i

        lane_is0 = lane == zeros_i

        def select_digit(kk):
            @plsc.parallel_loop(0, 16, unroll=4)
            def _(t):
                tot = zeros_i
                for l in range(_L):
                    tot = tot + hist_v[pl.ds(l * 256 + t * _L, _L)]
                    hist_v[pl.ds(l * 256 + t * _L, _L)] = zeros_i
                tot_v[pl.ds(t * _L, _L)] = tot
                s = jnp.full((_L,), jnp.sum(tot), jnp.int32)
                plsc.store_scatter(bs_v, [jnp.full((_L,), t, jnp.int32)], s,
                                   mask=lane_is0)

            bs = bs_v[pl.ds(0, _L)]
            sb = plsc.cumsum(lax.rev(bs, (0,)))
            blk_desc = jnp.int32(15) - lane
            cb = jnp.max(jnp.where(sb >= kk, blk_desc, jnp.int32(-1)))
            carry_above = jnp.sum(jnp.where(lane > cb, bs, zeros_i))
            tot_cb = tot_v[pl.ds(cb * _L, _L)]
            cs = plsc.cumsum(lax.rev(tot_cb, (0,))) + carry_above
            bin_desc = cb * jnp.int32(_L) + jnp.int32(15) - lane
            b_sel = jnp.max(jnp.where(cs >= kk, bin_desc, jnp.int32(-1)))
            bin_asc = cb * jnp.int32(_L) + lane
            cnt_gt = carry_above + jnp.sum(
                jnp.where(bin_asc > b_sel, tot_cb, zeros_i))
            return b_sel, kk - cnt_gt

        start_in(0)
        zero_hist()
        for j in range(rows_per_w):
            if j + 1 < rows_per_w:
                start_in(j + 1)
            in_handles[j].wait()
            boff = (j % 2) * C

            @plsc.parallel_loop(0, nvec, unroll=UNROLL)
            def _(i):
                v = rows_v[pl.ds(i * _L + boff, _L)]
                o = _to_ordered(v)
                d = jnp.right_shift(o, 24) + jnp.int32(128)
                plsc.addupdate_scatter(hist_v, [lane_base + d], ones_i)

            b0, kk = select_digit(kk0)
            prefix = b0 - jnp.int32(128)

            if j >= 1:
                out_handles[j - 1].wait()

            pref0 = prefix

            def compact_body(i, off):
                v = rows_v[pl.ds(i * _L + boff, _L)]
                o = _to_ordered(v)
                m = jnp.right_shift(o, 24) == pref0
                pos = plsc.cumsum(jnp.where(m, ones_i, zeros_i))
                plsc.store_scatter(aux_v, [off + pos],
                                   plsc.bitcast(o, jnp.float32), mask=m)
                return off + plsc.all_reduce_population_count(m)

            n1_splat = plsc.parallel_loop(
                0, nvec, unroll=UNROLL,
                carry=jnp.full((_L,), -1, jnp.int32))(compact_body)
            n1 = n1_splat[0] + jnp.int32(1)

            n_steps = jnp.right_shift(n1 + jnp.int32(_L - 1), 4)
            for p in (1, 2, 3):
                pref_s = prefix
                dsh = 24 - 8 * p

                @plsc.parallel_loop(0, n_steps, unroll=2)
                def _(i):
                    o = plsc.bitcast(aux_v[pl.ds(i * _L, _L)], jnp.int32)
                    m = (i * jnp.int32(_L) + lane) < n1
                    if p > 1:
                        m = m & (jnp.right_shift(o, 32 - 8 * p) == pref_s)
                    if dsh:
                        d = jnp.right_shift(o, dsh) & jnp.int32(0xFF)
                    else:
                        d = o & jnp.int32(0xFF)
                    plsc.addupdate_scatter(hist_v, [lane_base + d], ones_i,
                                           mask=m)

                b_sel, kk = select_digit(kk)
                prefix = prefix * jnp.int32(256) + b_sel

            thr_o = jnp.full((_L,), prefix, jnp.int32)
            thr_s = thr_o ^ (jnp.right_shift(thr_o, 31) & jnp.int32(_M31))
            thr_f = plsc.bitcast(thr_s, jnp.float32)

            @plsc.parallel_loop(0, nvec, unroll=UNROLL)
            def _(i):
                v = rows_v[pl.ds(i * _L + boff, _L)]
                aux_v[pl.ds(i * _L, _L)] = jnp.where(v >= thr_f, v, neg_f)

            out_handles[j] = pltpu.async_copy(
                aux_v, out_hbm.at[base_of(j)], sem_out)

        out_handles[rows_per_w - 1].wait()

    return sc_kernel


def kernel(scores, k):
    R, C = scores.shape
    k_arr = jnp.full((_L,), k, dtype=jnp.int32)
    return _build(R, C)(scores, k_arr)
